# baseline (device time: 101783 ns/iter reference)
import jax
import jax.numpy as jnp
from jax import lax
from jax.experimental import pallas as pl
from jax.experimental.pallas import tpu as pltpu

N_DEV = 8
E_PER_DEV = 2


def kernel(x, router_W, route_idx, expert_W, shared_W):
    n_tok, d_model = x.shape
    n_exp_total = router_W.shape[1]
    d_ff = expert_W.shape[2]

    def body(x_ref, rw_ref, idx_ref, ew_ref, sw_ref, out_ref,
             comm_ref, send_sems, recv_sems):
        my = lax.axis_index("i")
        left = lax.rem(my + N_DEV - 1, N_DEV)
        right = lax.rem(my + 1, N_DEV)

        barrier_sem = pltpu.get_barrier_semaphore()
        for nbr in (left, right):
            pl.semaphore_signal(
                barrier_sem, inc=1,
                device_id=(nbr,), device_id_type=pl.DeviceIdType.MESH,
            )
        pl.semaphore_wait(barrier_sem, 2)

        xv = x_ref[:, :]
        scores = jnp.dot(xv, rw_ref[:, :],
                         preferred_element_type=jnp.float32)
        s_max = jnp.max(scores, axis=-1, keepdims=True)
        p = jnp.exp(scores - s_max)
        probs = p / jnp.sum(p, axis=-1, keepdims=True)

        e_sel = idx_ref[:, :]
        iota_e = lax.broadcasted_iota(jnp.int32, (n_tok, n_exp_total), 1)
        p_routed = jnp.sum(
            jnp.where(iota_e == e_sel, probs, 0.0),
            axis=-1, keepdims=True)

        partial = jnp.zeros((n_tok, d_ff), jnp.float32)
        for j in range(E_PER_DEV):
            g = my * E_PER_DEV + j
            cj = jnp.where(e_sel == g, p_routed, 0.0)
            partial = partial + cj * jnp.dot(
                xv, ew_ref[j], preferred_element_type=jnp.float32)

        comm_ref[0] = partial
        out_ref[:, :] = partial + jnp.dot(
            xv, sw_ref[:, :], preferred_element_type=jnp.float32)

        for h in range(N_DEV - 1):
            rdma = pltpu.make_async_remote_copy(
                src_ref=comm_ref.at[h],
                dst_ref=comm_ref.at[h + 1],
                send_sem=send_sems.at[h],
                recv_sem=recv_sems.at[h],
                device_id=(right,),
                device_id_type=pl.DeviceIdType.MESH,
            )
            rdma.start()
            rdma.wait()
            out_ref[:, :] += comm_ref[h + 1]

    return pl.pallas_call(
        body,
        out_shape=jax.ShapeDtypeStruct((n_tok, d_ff), jnp.float32),
        in_specs=[
            pl.BlockSpec(memory_space=pltpu.VMEM),
            pl.BlockSpec(memory_space=pltpu.VMEM),
            pl.BlockSpec(memory_space=pltpu.VMEM),
            pl.BlockSpec(memory_space=pltpu.VMEM),
            pl.BlockSpec(memory_space=pltpu.VMEM),
        ],
        out_specs=pl.BlockSpec(memory_space=pltpu.VMEM),
        scratch_shapes=[
            pltpu.VMEM((N_DEV, n_tok, d_ff), jnp.float32),
            pltpu.SemaphoreType.DMA((N_DEV - 1,)),
            pltpu.SemaphoreType.DMA((N_DEV - 1,)),
        ],
        compiler_params=pltpu.CompilerParams(collective_id=0),
    )(x, router_W, route_idx, expert_W, shared_W)


# device time: 19880 ns/iter; 5.1199x vs baseline; 5.1199x over previous
import jax
import jax.numpy as jnp
from jax import lax
from jax.experimental import pallas as pl
from jax.experimental.pallas import tpu as pltpu

N_DEV = 8
E_PER_DEV = 2
N_HALF = 2


def kernel(x, router_W, route_idx, expert_W, shared_W):
    n_tok, d_model = x.shape
    n_exp_total = router_W.shape[1]
    d_ff = expert_W.shape[2]
    rows = n_tok // N_DEV
    hcol = d_ff // N_HALF

    def body(x_ref, rw_ref, idx_ref, ew_ref, sw_ref, out_ref,
             pbuf, staging, rbuf, stage2,
             a1_send, a1_recv, a2_send, a2_recv):
        my = lax.axis_index("i")

        barrier_sem = pltpu.get_barrier_semaphore()
        for d in range(1, N_DEV):
            pl.semaphore_signal(
                barrier_sem, inc=1,
                device_id=(lax.rem(my + d, N_DEV),),
                device_id_type=pl.DeviceIdType.MESH,
            )

        xv = x_ref[:, :]
        scores = jnp.dot(xv, rw_ref[:, :],
                         preferred_element_type=jnp.float32)
        s_max = jnp.max(scores, axis=-1, keepdims=True)
        p = jnp.exp(scores - s_max)
        probs = p / jnp.sum(p, axis=-1, keepdims=True)

        e_sel = idx_ref[:, :]
        iota_e = lax.broadcasted_iota(jnp.int32, (n_tok, n_exp_total), 1)
        p_routed = jnp.sum(
            jnp.where(iota_e == e_sel, probs, 0.0),
            axis=-1, keepdims=True)

        scaled = [
            jnp.where(e_sel == my * E_PER_DEV + j, p_routed, 0.0) * xv
            for j in range(E_PER_DEV)
        ]
        xcat = jnp.concatenate(scaled, axis=1).astype(jnp.bfloat16)
        wcat = ew_ref[:, :, :].reshape(
            E_PER_DEV * d_model, d_ff).astype(jnp.bfloat16)
        partial = jnp.dot(xcat, wcat, preferred_element_type=jnp.float32)
        pbuf[:, :] = partial.astype(jnp.bfloat16)

        pl.semaphore_wait(barrier_sem, N_DEV - 1)

        p1 = [[None] * N_DEV for _ in range(N_HALF)]
        for h in range(N_HALF):
            for d in range(1, N_DEV):
                peer = lax.rem(my + d, N_DEV)
                rdma = pltpu.make_async_remote_copy(
                    src_ref=pbuf.at[pl.ds(peer * rows, rows),
                                    pl.ds(h * hcol, hcol)],
                    dst_ref=staging.at[d, :, pl.ds(h * hcol, hcol)],
                    send_sem=a1_send.at[h, d],
                    recv_sem=a1_recv.at[h, d],
                    device_id=(peer,),
                    device_id_type=pl.DeviceIdType.MESH,
                )
                rdma.start()
                p1[h][d] = rdma

        xm = x_ref[pl.ds(my * rows, rows), :].astype(jnp.bfloat16)
        acc = jnp.dot(xm, sw_ref[:, :].astype(jnp.bfloat16),
                      preferred_element_type=jnp.float32)
        acc = acc + pbuf[pl.ds(my * rows, rows), :].astype(jnp.float32)

        p2 = [[None] * N_DEV for _ in range(N_HALF)]
        for h in range(N_HALF):
            acc_h = acc[:, h * hcol:(h + 1) * hcol]
            for d in range(1, N_DEV):
                p1[h][d].wait_recv()
                acc_h = acc_h + staging[
                    d, :, pl.ds(h * hcol, hcol)].astype(jnp.float32)
            rbuf[:, pl.ds(h * hcol, hcol)] = acc_h.astype(jnp.bfloat16)
            out_ref[pl.ds(my * rows, rows), pl.ds(h * hcol, hcol)] = acc_h
            for d in range(1, N_DEV):
                peer = lax.rem(my + d, N_DEV)
                rdma = pltpu.make_async_remote_copy(
                    src_ref=rbuf.at[:, pl.ds(h * hcol, hcol)],
                    dst_ref=stage2.at[d, :, pl.ds(h * hcol, hcol)],
                    send_sem=a2_send.at[h, d],
                    recv_sem=a2_recv.at[h, d],
                    device_id=(peer,),
                    device_id_type=pl.DeviceIdType.MESH,
                )
                rdma.start()
                p2[h][d] = rdma

        for h in range(N_HALF):
            for d in range(1, N_DEV):
                p2[h][d].wait_recv()
                src_rank = lax.rem(my + N_DEV - d, N_DEV)
                out_ref[pl.ds(src_rank * rows, rows),
                        pl.ds(h * hcol, hcol)] = (
                    stage2[d, :, pl.ds(h * hcol, hcol)].astype(jnp.float32))
        for h in range(N_HALF):
            for d in range(1, N_DEV):
                p1[h][d].wait_send()
                p2[h][d].wait_send()

    return pl.pallas_call(
        body,
        out_shape=jax.ShapeDtypeStruct((n_tok, d_ff), jnp.float32),
        in_specs=[
            pl.BlockSpec(memory_space=pltpu.VMEM),
            pl.BlockSpec(memory_space=pltpu.VMEM),
            pl.BlockSpec(memory_space=pltpu.VMEM),
            pl.BlockSpec(memory_space=pltpu.VMEM),
            pl.BlockSpec(memory_space=pltpu.VMEM),
        ],
        out_specs=pl.BlockSpec(memory_space=pltpu.VMEM),
        scratch_shapes=[
            pltpu.VMEM((n_tok, d_ff), jnp.bfloat16),
            pltpu.VMEM((N_DEV, rows, d_ff), jnp.bfloat16),
            pltpu.VMEM((rows, d_ff), jnp.bfloat16),
            pltpu.VMEM((N_DEV, rows, d_ff), jnp.bfloat16),
            pltpu.SemaphoreType.DMA((N_HALF, N_DEV)),
            pltpu.SemaphoreType.DMA((N_HALF, N_DEV)),
            pltpu.SemaphoreType.DMA((N_HALF, N_DEV)),
            pltpu.SemaphoreType.DMA((N_HALF, N_DEV)),
        ],
        compiler_params=pltpu.CompilerParams(collective_id=0),
    )(x, router_W, route_idx, expert_W, shared_W)
